# baseline (speedup 1.0000x reference)
import functools

import jax
import jax.numpy as jnp
from jax import lax
from jax.experimental import pallas as pl
from jax.experimental.pallas import tpu as pltpu
from jax.experimental.pallas import tpu_sc as plsc

N_NODES = 10000
N_EDGES = 160000
CHANNELS = 256
HALF = CHANNELS // 2
NSUB = 16
NBUF = 4
CHUNK = 96
CPS = 108
NOUTER = CPS // NBUF
EPT = CPS * CHUNK
E_PAD = EPT * NSUB
NCHUNK_TOT = E_PAD // CHUNK
ROWS_PER_SUB = N_NODES // NSUB
DUMMY = N_NODES


def _sc_agg(x2, s2, t2):
  mesh = plsc.VectorSubcoreMesh(core_axis_name="c", subcore_axis_name="s")

  @functools.partial(
      pl.kernel,
      out_type=jax.ShapeDtypeStruct((N_NODES, CHANNELS), jnp.float32),
      mesh=mesh,
      scratch_types=[
          pltpu.VMEM((2, NBUF, CHUNK), jnp.int32),
          pltpu.VMEM((2, NBUF, CHUNK), jnp.int32),
          pltpu.VMEM((NBUF, CHUNK, HALF), jnp.float32),
          pltpu.VMEM_SHARED((N_NODES + 8, HALF), jnp.float32),
          pltpu.SemaphoreType.DMA,
          pltpu.SemaphoreType.DMA,
          pltpu.SemaphoreType.DMA,
          pltpu.SemaphoreType.DMA,
          pltpu.SemaphoreType.DMA,
      ],
      compiler_params=pltpu.CompilerParams(use_tc_tiling_on_sc=False),
  )
  def k(x2_hbm, s_hbm, t_hbm, out_hbm, sblk, tblk, rows, slab,
        g0, g1, g2, g3, gi):
    gsem = [g0, g1, g2, g3]
    c = lax.axis_index("c")
    s = lax.axis_index("s")
    my_x = x2_hbm.at[c]

    pltpu.sync_copy(
        my_x.at[pl.ds(s * ROWS_PER_SUB, ROWS_PER_SUB)],
        slab.at[pl.ds(s * ROWS_PER_SUB, ROWS_PER_SUB)],
    )
    plsc.subcore_barrier()

    cbase = s * CPS

    pltpu.sync_copy(s_hbm.at[pl.ds(cbase, NBUF)], sblk.at[0])
    pltpu.sync_copy(t_hbm.at[pl.ds(cbase, NBUF)], tblk.at[0])

    def outer(g):
      gp = g % 2
      for b in range(NBUF):
        pltpu.async_copy(my_x.at[sblk.at[gp, b]], rows.at[b], gsem[b])

      @pl.when(g + 1 < NOUTER)
      def _pref():
        off = cbase + (g + 1) * NBUF
        pltpu.async_copy(s_hbm.at[pl.ds(off, NBUF)], sblk.at[1 - gp], gi)
        pltpu.async_copy(t_hbm.at[pl.ds(off, NBUF)], tblk.at[1 - gp], gi)

      for b in range(NBUF):
        pltpu.make_async_copy(my_x.at[sblk.at[gp, b]], rows.at[b],
                              gsem[b]).wait()
        pltpu.sync_copy(rows.at[b], slab.at[tblk.at[gp, b]], add=True)

      @pl.when(g + 1 < NOUTER)
      def _wait_pref():
        off = cbase + (g + 1) * NBUF
        pltpu.make_async_copy(s_hbm.at[pl.ds(off, NBUF)], sblk.at[1 - gp],
                              gi).wait()
        pltpu.make_async_copy(t_hbm.at[pl.ds(off, NBUF)], tblk.at[1 - gp],
                              gi).wait()

    pl.loop(0, NOUTER)(outer)
    plsc.subcore_barrier()

    pltpu.sync_copy(
        slab.at[pl.ds(s * ROWS_PER_SUB, ROWS_PER_SUB)],
        out_hbm.at[pl.ds(s * ROWS_PER_SUB, ROWS_PER_SUB), pl.ds(c * HALF, HALF)],
    )

  return k(x2, s2, t2)


def _mm_body(agg_ref, norm_ref, w_ref, out_ref):
  h = norm_ref[...] * agg_ref[...]
  out_ref[...] = jnp.dot(h, w_ref[...], preferred_element_type=jnp.float32)


def _tc_matmul(agg, norm, weight):
  bn = 512
  grid = (pl.cdiv(N_NODES, bn),)
  return pl.pallas_call(
      _mm_body,
      grid=grid,
      in_specs=[
          pl.BlockSpec((bn, CHANNELS), lambda i: (i, 0)),
          pl.BlockSpec((bn, 1), lambda i: (i, 0)),
          pl.BlockSpec((CHANNELS, CHANNELS), lambda i: (0, 0)),
      ],
      out_specs=pl.BlockSpec((bn, CHANNELS), lambda i: (i, 0)),
      out_shape=jax.ShapeDtypeStruct((N_NODES, CHANNELS), jnp.float32),
  )(agg, norm, weight)


def kernel(x, sources, targets, norm, weight):
  pad = E_PAD - N_EDGES
  s32 = jnp.concatenate(
      [sources.astype(jnp.int32), jnp.zeros((pad,), jnp.int32)])
  t32 = jnp.concatenate(
      [targets.astype(jnp.int32), jnp.full((pad,), DUMMY, jnp.int32)])
  s2 = s32.reshape(NCHUNK_TOT, CHUNK)
  t2 = t32.reshape(NCHUNK_TOT, CHUNK)
  x2 = x.reshape(N_NODES, 2, HALF).transpose(1, 0, 2)
  agg = _sc_agg(x2, s2, t2)
  return _tc_matmul(agg, norm, weight)


# speedup vs baseline: 2.2711x; 2.2711x over previous
import functools

import jax
import jax.numpy as jnp
from jax import lax
from jax.experimental import pallas as pl
from jax.experimental.pallas import tpu as pltpu
from jax.experimental.pallas import tpu_sc as plsc

N_NODES = 10000
N_EDGES = 160000
CHANNELS = 256
QUART = CHANNELS // 4
NSUB = 16
NBUF = 4
CHUNK = 96
CPS = 108
NOUTER = CPS // NBUF
EPT = CPS * CHUNK
E_PAD = EPT * NSUB
NCHUNK_TOT = E_PAD // CHUNK
ROWS_PER_SUB = N_NODES // NSUB
DUMMY = N_NODES


def _sc_agg(x4, s2, t2):
  mesh = plsc.VectorSubcoreMesh(core_axis_name="c", subcore_axis_name="s")

  @functools.partial(
      pl.kernel,
      out_type=jax.ShapeDtypeStruct((N_NODES, CHANNELS), jnp.float32),
      mesh=mesh,
      scratch_types=[
          pltpu.VMEM((2, NBUF, CHUNK), jnp.int32),
          pltpu.VMEM((2, NBUF, CHUNK), jnp.int32),
          pltpu.VMEM((NBUF, CHUNK, QUART), jnp.float32),
          pltpu.VMEM_SHARED((N_NODES, QUART), jnp.float32),
          pltpu.VMEM_SHARED((N_NODES + 8, QUART), jnp.float32),
          pltpu.SemaphoreType.DMA,
          pltpu.SemaphoreType.DMA,
          pltpu.SemaphoreType.DMA,
          pltpu.SemaphoreType.DMA,
          pltpu.SemaphoreType.DMA,
      ],
      compiler_params=pltpu.CompilerParams(use_tc_tiling_on_sc=False),
  )
  def k(x4_hbm, s_hbm, t_hbm, out_hbm, sblk, tblk, rows, table, slab,
        g0, g1, g2, g3, gi):
    gsem = [g0, g1, g2, g3]
    c = lax.axis_index("c")
    s = lax.axis_index("s")
    nsl = pl.ds(s * ROWS_PER_SUB, ROWS_PER_SUB)
    cbase = s * CPS

    for p in range(2):
      q = c * 2 + p
      my_x = x4_hbm.at[q]

      pltpu.sync_copy(my_x.at[nsl], table.at[nsl])
      pltpu.sync_copy(my_x.at[nsl], slab.at[nsl])
      pltpu.sync_copy(s_hbm.at[pl.ds(cbase, NBUF)], sblk.at[0])
      pltpu.sync_copy(t_hbm.at[pl.ds(cbase, NBUF)], tblk.at[0])
      plsc.subcore_barrier()

      def outer(g):
        gp = g % 2
        for b in range(NBUF):
          pltpu.async_copy(table.at[sblk.at[gp, b]], rows.at[b], gsem[b])

        @pl.when(g + 1 < NOUTER)
        def _pref():
          off = cbase + (g + 1) * NBUF
          pltpu.async_copy(s_hbm.at[pl.ds(off, NBUF)], sblk.at[1 - gp], gi)
          pltpu.async_copy(t_hbm.at[pl.ds(off, NBUF)], tblk.at[1 - gp], gi)

        for b in range(NBUF):
          pltpu.make_async_copy(table.at[sblk.at[gp, b]], rows.at[b],
                                gsem[b]).wait()
          pltpu.sync_copy(rows.at[b], slab.at[tblk.at[gp, b]], add=True)

        @pl.when(g + 1 < NOUTER)
        def _wait_pref():
          off = cbase + (g + 1) * NBUF
          pltpu.make_async_copy(s_hbm.at[pl.ds(off, NBUF)], sblk.at[1 - gp],
                                gi).wait()
          pltpu.make_async_copy(t_hbm.at[pl.ds(off, NBUF)], tblk.at[1 - gp],
                                gi).wait()

      pl.loop(0, NOUTER)(outer)
      plsc.subcore_barrier()

      pltpu.sync_copy(
          slab.at[nsl],
          out_hbm.at[nsl, pl.ds(q * QUART, QUART)],
      )
      plsc.subcore_barrier()

  return k(x4, s2, t2)


def _mm_body(agg_ref, norm_ref, w_ref, out_ref):
  h = norm_ref[...] * agg_ref[...]
  out_ref[...] = jnp.dot(h, w_ref[...], preferred_element_type=jnp.float32)


def _tc_matmul(agg, norm, weight):
  bn = 512
  grid = (pl.cdiv(N_NODES, bn),)
  return pl.pallas_call(
      _mm_body,
      grid=grid,
      in_specs=[
          pl.BlockSpec((bn, CHANNELS), lambda i: (i, 0)),
          pl.BlockSpec((bn, 1), lambda i: (i, 0)),
          pl.BlockSpec((CHANNELS, CHANNELS), lambda i: (0, 0)),
      ],
      out_specs=pl.BlockSpec((bn, CHANNELS), lambda i: (i, 0)),
      out_shape=jax.ShapeDtypeStruct((N_NODES, CHANNELS), jnp.float32),
  )(agg, norm, weight)


def kernel(x, sources, targets, norm, weight):
  pad = E_PAD - N_EDGES
  s32 = jnp.concatenate(
      [sources.astype(jnp.int32), jnp.zeros((pad,), jnp.int32)])
  t32 = jnp.concatenate(
      [targets.astype(jnp.int32), jnp.full((pad,), DUMMY, jnp.int32)])
  s2 = s32.reshape(NCHUNK_TOT, CHUNK)
  t2 = t32.reshape(NCHUNK_TOT, CHUNK)
  x4 = x.reshape(N_NODES, 4, QUART).transpose(1, 0, 2)
  agg = _sc_agg(x4, s2, t2)
  return _tc_matmul(agg, norm, weight)


# speedup vs baseline: 2.5112x; 1.1057x over previous
import functools

import jax
import jax.numpy as jnp
from jax import lax
from jax.experimental import pallas as pl
from jax.experimental.pallas import tpu as pltpu
from jax.experimental.pallas import tpu_sc as plsc

N_NODES = 10000
N_EDGES = 160000
CHANNELS = 256
QUART = CHANNELS // 4
NSUB = 16
NBUF = 4
CHUNK = 96
CPS = 108
NOUTER = CPS // NBUF
EPT = CPS * CHUNK
E_PAD = EPT * NSUB
NCHUNK_TOT = E_PAD // CHUNK
ROWS_PER_SUB = N_NODES // NSUB
DUMMY = N_NODES


def _sc_agg(x, s2, t2):
  mesh = plsc.VectorSubcoreMesh(core_axis_name="c", subcore_axis_name="s")

  @functools.partial(
      pl.kernel,
      out_type=jax.ShapeDtypeStruct((N_NODES, CHANNELS), jnp.float32),
      mesh=mesh,
      scratch_types=[
          pltpu.VMEM((CPS, CHUNK), jnp.int32),
          pltpu.VMEM((CPS, CHUNK), jnp.int32),
          pltpu.VMEM((NBUF, CHUNK, QUART), jnp.float32),
          pltpu.VMEM_SHARED((N_NODES, QUART), jnp.float32),
          pltpu.VMEM_SHARED((N_NODES + 8, QUART), jnp.float32),
          pltpu.SemaphoreType.DMA,
          pltpu.SemaphoreType.DMA,
          pltpu.SemaphoreType.DMA,
          pltpu.SemaphoreType.DMA,
      ],
      compiler_params=pltpu.CompilerParams(use_tc_tiling_on_sc=False),
  )
  def k(x_hbm, s_hbm, t_hbm, out_hbm, sidx, tidx, rows, table, slab,
        g0, g1, g2, g3):
    gsem = [g0, g1, g2, g3]
    c = lax.axis_index("c")
    s = lax.axis_index("s")
    nsl = pl.ds(s * ROWS_PER_SUB, ROWS_PER_SUB)
    cbase = s * CPS

    pltpu.sync_copy(s_hbm.at[pl.ds(cbase, CPS)], sidx)
    pltpu.sync_copy(t_hbm.at[pl.ds(cbase, CPS)], tidx)

    for p in range(2):
      q = c * 2 + p
      csl = pl.ds(q * QUART, QUART)

      pltpu.sync_copy(x_hbm.at[nsl, csl], table.at[nsl])
      pltpu.sync_copy(x_hbm.at[nsl, csl], slab.at[nsl])
      plsc.subcore_barrier()

      def outer(g):
        for b in range(NBUF):
          pltpu.async_copy(table.at[sidx.at[g * NBUF + b]], rows.at[b],
                           gsem[b])
        for b in range(NBUF):
          pltpu.make_async_copy(table.at[sidx.at[g * NBUF + b]], rows.at[b],
                                gsem[b]).wait()
          pltpu.sync_copy(rows.at[b], slab.at[tidx.at[g * NBUF + b]],
                          add=True)

      pl.loop(0, NOUTER)(outer)
      plsc.subcore_barrier()

      pltpu.sync_copy(slab.at[nsl], out_hbm.at[nsl, csl])
      plsc.subcore_barrier()

  return k(x, s2, t2)


def _mm_body(agg_ref, norm_ref, w_ref, out_ref):
  h = norm_ref[...] * agg_ref[...]
  out_ref[...] = jnp.dot(h, w_ref[...], preferred_element_type=jnp.float32)


def _tc_matmul(agg, norm, weight):
  bn = 512
  grid = (pl.cdiv(N_NODES, bn),)
  return pl.pallas_call(
      _mm_body,
      grid=grid,
      in_specs=[
          pl.BlockSpec((bn, CHANNELS), lambda i: (i, 0)),
          pl.BlockSpec((bn, 1), lambda i: (i, 0)),
          pl.BlockSpec((CHANNELS, CHANNELS), lambda i: (0, 0)),
      ],
      out_specs=pl.BlockSpec((bn, CHANNELS), lambda i: (i, 0)),
      out_shape=jax.ShapeDtypeStruct((N_NODES, CHANNELS), jnp.float32),
  )(agg, norm, weight)


def kernel(x, sources, targets, norm, weight):
  pad = E_PAD - N_EDGES
  s32 = jnp.concatenate(
      [sources.astype(jnp.int32), jnp.zeros((pad,), jnp.int32)])
  t32 = jnp.concatenate(
      [targets.astype(jnp.int32), jnp.full((pad,), DUMMY, jnp.int32)])
  s2 = s32.reshape(NCHUNK_TOT, CHUNK)
  t2 = t32.reshape(NCHUNK_TOT, CHUNK)
  agg = _sc_agg(x, s2, t2)
  return _tc_matmul(agg, norm, weight)
